# baseline (speedup 1.0000x reference)
import functools

import jax
import jax.numpy as jnp
from jax import lax
from jax.experimental import pallas as pl
from jax.experimental.pallas import tpu as pltpu
from jax.experimental.pallas import tpu_sc as plsc

B, N, D, M = 16, 1024, 768, 300
LANES = 16
LW = 176


def _sc_gather(seg_flat, idx_flat):
    mesh = plsc.VectorSubcoreMesh(core_axis_name="c", subcore_axis_name="s")

    @functools.partial(
        pl.kernel,
        mesh=mesh,
        out_type=jax.ShapeDtypeStruct((B, M, D), jnp.float32),
        scratch_types=[
            pltpu.VMEM((LW,), jnp.int32),
            pltpu.VMEM((LW,), jnp.int32),
            pltpu.VMEM((LANES,), jnp.int32),
            pltpu.VMEM((160, D), jnp.float32),
            pltpu.SemaphoreType.DMA,
        ],
    )
    def k(seg_hbm, idx_hbm, out_hbm, raw_v, glob_v, didx_v, rows_v, sem):
        wid = lax.axis_index("s") * 2 + lax.axis_index("c")
        b = wid // 2
        half = wid % 2
        row_off = b * N
        start = b * M + half * 160
        d = start % 8
        s0 = pl.multiple_of(start - d, 8)
        iot = lax.iota(jnp.int32, 16)

        def vtake(v, i):
            dn = lax.GatherDimensionNumbers(
                offset_dims=(), collapsed_slice_dims=(0,), start_index_map=(0,)
            )
            return lax.gather(
                v, i[:, None], dn, (1,),
                mode=lax.GatherScatterMode.PROMISE_IN_BOUNDS,
            )

        def shifted(joff, dd):
            a = raw_v[pl.ds(joff, LANES)]
            bn = raw_v[pl.ds(joff + LANES, LANES)]
            ia = jnp.minimum(iot + dd, LANES - 1)
            ib = jnp.maximum(iot + dd - LANES, 0)
            return jnp.where(iot < LANES - dd, vtake(a, ia), vtake(bn, ib))

        @pl.when(half == 0)
        def _():
            pltpu.sync_copy(idx_hbm.at[pl.ds(s0, LW)], raw_v)
            for j in range(10):
                glob_v[pl.ds(j * LANES, LANES)] = shifted(j * LANES, d) + row_off
            for c in range(2):
                pltpu.async_copy(
                    seg_hbm.at[glob_v.at[pl.ds(c * 80, 80)]],
                    rows_v.at[pl.ds(c * 80, 80)],
                    sem,
                ).wait()
            pltpu.sync_copy(rows_v, out_hbm.at[b, pl.ds(0, 160), :])

        @pl.when(half == 1)
        def _():
            pltpu.sync_copy(idx_hbm.at[pl.ds(s0, 144)], raw_v.at[pl.ds(0, 144)])
            for j in range(9):
                g = shifted(j * LANES, d) + row_off
                if j == 8:
                    g = jnp.clip(g, 0, B * N - 1)
                glob_v[pl.ds(j * LANES, LANES)] = g
            didx_v[...] = 284 + iot
            tglob = shifted(112, d + 12) + row_off
            glob_v[pl.ds(160, LANES)] = tglob
            pltpu.async_copy(
                seg_hbm.at[glob_v.at[pl.ds(0, 80)]],
                rows_v.at[pl.ds(0, 80)],
                sem,
            ).wait()
            pltpu.async_copy(
                seg_hbm.at[glob_v.at[pl.ds(80, 64)]],
                rows_v.at[pl.ds(80, 64)],
                sem,
            ).wait()
            pltpu.async_copy(
                seg_hbm.at[glob_v.at[pl.ds(160, LANES)]],
                rows_v.at[pl.ds(144, LANES)],
                sem,
            ).wait()
            pltpu.sync_copy(
                rows_v.at[pl.ds(0, 136)], out_hbm.at[b, pl.ds(160, 136), :]
            )
            pltpu.async_copy(
                rows_v.at[pl.ds(144, LANES)],
                out_hbm.at[b].at[didx_v],
                sem,
            ).wait()

    return k(seg_flat, idx_flat)


def kernel(seg_out, det_out, det_scores, det_indices):
    idx = det_indices.astype(jnp.int32).reshape(B * M)
    return _sc_gather(seg_out.reshape(B * N, D), idx)


# speedup vs baseline: 1.0398x; 1.0398x over previous
import functools

import jax
import jax.numpy as jnp
from jax import lax
from jax.experimental import pallas as pl
from jax.experimental.pallas import tpu as pltpu
from jax.experimental.pallas import tpu_sc as plsc

B, N, D, M = 16, 1024, 768, 300
MP = 304
LANES = 16


def _sc_gather(seg_flat, idx_flat):
    mesh = plsc.VectorSubcoreMesh(core_axis_name="c", subcore_axis_name="s")

    @functools.partial(
        pl.kernel,
        mesh=mesh,
        out_type=jax.ShapeDtypeStruct((B, M, D), jnp.float32),
        scratch_types=[
            pltpu.VMEM((160,), jnp.int32),
            pltpu.VMEM((LANES,), jnp.int32),
            pltpu.VMEM((160, D), jnp.float32),
            pltpu.SemaphoreType.DMA,
            pltpu.SemaphoreType.DMA,
        ],
    )
    def k(seg_hbm, idx_hbm, out_hbm, idx_v, didx_v, rows_v, sem_g, sem_w):
        wid = lax.axis_index("s") * 2 + lax.axis_index("c")
        b = wid // 2
        half = wid % 2
        row_off = b * N
        iot = lax.iota(jnp.int32, 16)

        @pl.when(half == 0)
        def _():
            pltpu.sync_copy(idx_hbm.at[pl.ds(b * MP, 160)], idx_v)
            for j in range(10):
                sl = pl.ds(j * LANES, LANES)
                idx_v[sl] = idx_v[sl] + row_off
            g0 = pltpu.async_copy(
                seg_hbm.at[idx_v.at[pl.ds(0, 80)]],
                rows_v.at[pl.ds(0, 80)], sem_g)
            g1 = pltpu.async_copy(
                seg_hbm.at[idx_v.at[pl.ds(80, 80)]],
                rows_v.at[pl.ds(80, 80)], sem_g)
            g0.wait()
            w0 = pltpu.async_copy(
                rows_v.at[pl.ds(0, 80)], out_hbm.at[b, pl.ds(0, 80), :], sem_w)
            g1.wait()
            w1 = pltpu.async_copy(
                rows_v.at[pl.ds(80, 80)], out_hbm.at[b, pl.ds(80, 80), :], sem_w)
            w0.wait()
            w1.wait()

        @pl.when(half == 1)
        def _():
            pltpu.sync_copy(
                idx_hbm.at[pl.ds(b * MP + 160, 144)], idx_v.at[pl.ds(0, 144)]
            )
            for j in range(9):
                sl = pl.ds(j * LANES, LANES)
                idx_v[sl] = idx_v[sl] + row_off
            didx_v[...] = jnp.minimum(288 + iot, M - 1)
            g0 = pltpu.async_copy(
                seg_hbm.at[idx_v.at[pl.ds(0, 80)]],
                rows_v.at[pl.ds(0, 80)], sem_g)
            g1 = pltpu.async_copy(
                seg_hbm.at[idx_v.at[pl.ds(80, 64)]],
                rows_v.at[pl.ds(80, 64)], sem_g)
            g0.wait()
            w0 = pltpu.async_copy(
                rows_v.at[pl.ds(0, 80)], out_hbm.at[b, pl.ds(160, 80), :], sem_w)
            g1.wait()
            w1 = pltpu.async_copy(
                rows_v.at[pl.ds(80, 56)], out_hbm.at[b, pl.ds(240, 56), :], sem_w)
            wt = pltpu.async_copy(
                rows_v.at[pl.ds(128, LANES)], out_hbm.at[b].at[didx_v], sem_w)
            w0.wait()
            w1.wait()
            wt.wait()

    return k(seg_flat, idx_flat)


def kernel(seg_out, det_out, det_scores, det_indices):
    idx = det_indices.astype(jnp.int32)
    idx = jnp.pad(idx, ((0, 0), (0, MP - M)), mode="edge")
    return _sc_gather(seg_out.reshape(B * N, D), idx.reshape(B * MP))
